# baseline (speedup 1.0000x reference)
import functools

import jax
import jax.numpy as jnp
from jax import lax
from jax.experimental import pallas as pl
from jax.experimental.pallas import tpu as pltpu
from jax.experimental.pallas import tpu_sc as plsc

N = 10000
E = 320000
D_IN = 128
D = 64
G = 64
STEPS = 3

NC = 2
NS = 16
NW = NC * NS
CHUNK = 128
CHUNKS_PER_W = 80
HALF_CHUNKS = CHUNKS_PER_W // 2
NPAIR = HALF_CHUNKS // 2
E_PAD = NW * CHUNK * CHUNKS_PER_W
EDGES_PER_W = CHUNK * CHUNKS_PER_W
N_ACC = 10112
INIT_ROWS = N_ACC // NS
OUT_ROWS = N_ACC // NS

BLK = 1000
NBLK = N // BLK


def _dot(a, b):
    return jnp.dot(a, b, preferred_element_type=jnp.float32)


def _dot_hi(a, b):
    return jnp.dot(a, b, preferred_element_type=jnp.float32,
                   precision=lax.Precision.HIGHEST)


def _dot_t_hi(a, b):
    return lax.dot_general(a, b, (((0,), (0,)), ((), ())),
                           preferred_element_type=jnp.float32,
                           precision=lax.Precision.HIGHEST)


def _make_sc_body(width):
    def body(tab_hbm, zeros_hbm, src_hbm, dst_hbm, out_hbm,
             acc, src_v, dst_v, rows0, rows1, semg0, semg1):
        c = lax.axis_index("c")
        s = lax.axis_index("s")
        wid = c * NS + s
        pltpu.sync_copy(zeros_hbm.at[pl.ds(s * INIT_ROWS, INIT_ROWS)],
                        acc.at[pl.ds(s * INIT_ROWS, INIT_ROWS)])
        plsc.subcore_barrier()

        for half in range(2):
            hbase = wid * CHUNKS_PER_W + half * HALF_CHUNKS
            pltpu.sync_copy(src_hbm.at[pl.ds(hbase, HALF_CHUNKS)], src_v)
            pltpu.sync_copy(dst_hbm.at[pl.ds(hbase, HALF_CHUNKS)], dst_v)

            pltpu.async_copy(tab_hbm.at[src_v.at[0]], rows0, semg0)

            def pair(kk, carry):
                a = 2 * kk
                pltpu.make_async_copy(tab_hbm.at[src_v.at[a]], rows0,
                                      semg0).wait()
                sc_a = pltpu.async_copy(rows0, acc.at[dst_v.at[a]], semg0,
                                        add=True)
                g_b = pltpu.async_copy(tab_hbm.at[src_v.at[a + 1]], rows1,
                                       semg1)
                sc_a.wait()

                @pl.when(kk < NPAIR - 1)
                def _():
                    pltpu.async_copy(tab_hbm.at[src_v.at[a + 2]], rows0,
                                     semg0)

                g_b.wait()
                pltpu.async_copy(rows1, acc.at[dst_v.at[a + 1]], semg1,
                                 add=True).wait()
                return carry

            lax.fori_loop(0, NPAIR, pair, 0)
        plsc.subcore_barrier()
        pltpu.sync_copy(acc.at[pl.ds(s * OUT_ROWS, OUT_ROWS)],
                        out_hbm.at[c].at[pl.ds(s * OUT_ROWS, OUT_ROWS)])

    return body


@functools.cache
def _get_sc_scatter(width):
    return pl.kernel(
        _make_sc_body(width),
        out_type=jax.ShapeDtypeStruct((NC, N_ACC, width), jnp.float32),
        mesh=plsc.VectorSubcoreMesh(core_axis_name="c", subcore_axis_name="s",
                                    num_cores=NC, num_subcores=NS),
        scratch_types=[
            pltpu.VMEM_SHARED((N_ACC, width), jnp.float32),
            pltpu.VMEM((HALF_CHUNKS, CHUNK), jnp.int32),
            pltpu.VMEM((HALF_CHUNKS, CHUNK), jnp.int32),
            pltpu.VMEM((CHUNK, width), jnp.float32),
            pltpu.VMEM((CHUNK, width), jnp.float32),
            pltpu.SemaphoreType.DMA,
            pltpu.SemaphoreType.DMA,
        ],
        compiler_params=pltpu.CompilerParams(use_tc_tiling_on_sc=False),
    )


def _tc_l1_body(agg_ref, x_ref, wr_ref, wo_ref, b_ref, h_ref):
    aggs = agg_ref[0] + agg_ref[1]
    h_ref[...] = jnp.maximum(
        _dot(aggs, wr_ref[...]) + _dot(x_ref[...], wo_ref[...]) + b_ref[...],
        0.0)


_tc_l1 = pl.pallas_call(
    _tc_l1_body,
    grid=(NBLK,),
    in_specs=[
        pl.BlockSpec((2, BLK, D_IN), lambda i: (0, i, 0)),
        pl.BlockSpec((BLK, D_IN), lambda i: (i, 0)),
        pl.BlockSpec((D_IN, D), lambda i: (0, 0)),
        pl.BlockSpec((D_IN, D), lambda i: (0, 0)),
        pl.BlockSpec((1, D), lambda i: (0, 0)),
    ],
    out_specs=pl.BlockSpec((BLK, D), lambda i: (i, 0)),
    out_shape=jax.ShapeDtypeStruct((N, D), jnp.float32),
)


def _tc_post_body(agg_ref, h1_ref, batch_ref, wr_ref, wo2_ref, b2_ref,
                  wih_ref, whh_ref, bih_ref, bhh_ref, wd_ref, bd_ref,
                  wo_ref, bo_ref, out_ref, h_s):
    seg = lax.broadcasted_iota(jnp.int32, (1, G), 1)
    bg = bih_ref[...] + bhh_ref[...]

    def init_blk(i, carry):
        blk = pl.ds(i * BLK, BLK)
        aggs = agg_ref[0, blk, :] + agg_ref[1, blk, :]
        h_s[blk, :] = jnp.maximum(
            _dot(aggs, wr_ref[...]) + _dot(h1_ref[blk, :], wo2_ref[...])
            + b2_ref[...], 0.0)
        return carry

    lax.fori_loop(0, NBLK, init_blk, 0)

    q_star = jnp.zeros((G, 2 * D), jnp.float32)
    hh = jnp.zeros((G, D), jnp.float32)
    cc = jnp.zeros((G, D), jnp.float32)
    for _ in range(STEPS):
        gates = _dot(q_star, wih_ref[...]) + _dot(hh, whh_ref[...]) + bg
        gi = jax.nn.sigmoid(gates[:, 0:D])
        gf = jax.nn.sigmoid(gates[:, D:2 * D])
        gg = jnp.tanh(gates[:, 2 * D:3 * D])
        go = jax.nn.sigmoid(gates[:, 3 * D:4 * D])
        cc = gf * cc + gi * gg
        hh = go * jnp.tanh(cc)

        def pass_a(i, emax_c):
            blk = pl.ds(i * BLK, BLK)
            mb = batch_ref[blk, :] == seg
            qb = _dot_hi(mb.astype(jnp.float32), hh)
            e = jnp.sum(h_s[blk, :] * qb, axis=1, keepdims=True)
            blkmax = jnp.max(jnp.where(mb, e, -jnp.inf), axis=0,
                             keepdims=True)
            return jnp.maximum(emax_c, blkmax)

        emax = lax.fori_loop(0, NBLK, pass_a,
                             jnp.full((1, G), -jnp.inf, jnp.float32))

        def pass_b(i, carry):
            den_c, run_c = carry
            blk = pl.ds(i * BLK, BLK)
            mb = batch_ref[blk, :] == seg
            m = mb.astype(jnp.float32)
            hb = h_s[blk, :]
            qb = _dot_hi(m, hh)
            e = jnp.sum(hb * qb, axis=1, keepdims=True)
            emaxb = jnp.sum(jnp.where(mb, emax, 0.0), axis=1, keepdims=True)
            eexp = jnp.exp(e - emaxb)
            den_c = den_c + _dot_t_hi(m, eexp)
            run_c = run_c + _dot_t_hi(m, eexp * hb)
            return den_c, run_c

        den, run = lax.fori_loop(
            0, NBLK, pass_b,
            (jnp.zeros((G, 1), jnp.float32), jnp.zeros((G, D), jnp.float32)))
        r = run / jnp.where(den > 0.0, den, 1.0)
        q_star = jnp.concatenate([hh, r], axis=1)

    o = jnp.maximum(_dot(q_star, wd_ref[...]) + bd_ref[...], 0.0)
    out_ref[...] = _dot(o, wo_ref[...]) + bo_ref[...]


_tc_post = pl.pallas_call(
    _tc_post_body,
    out_shape=jax.ShapeDtypeStruct((G, 1), jnp.float32),
    scratch_shapes=[pltpu.VMEM((N, D), jnp.float32)],
)


def kernel(x, edge_index, batch, W_rel1, W_root1, b1, W_rel2, W_root2, b2,
           W_ih, W_hh, b_ih, b_hh, W_dense, b_dense, W_out, b_out):
    src = edge_index[0].astype(jnp.int32)
    dst = edge_index[1].astype(jnp.int32)
    pad = E_PAD - E
    src_pad = jnp.concatenate([src, jnp.zeros((pad,), jnp.int32)]).reshape(
        NW * CHUNKS_PER_W, CHUNK)
    dst_pad = jnp.concatenate([dst, jnp.full((pad,), N, jnp.int32)]).reshape(
        NW * CHUNKS_PER_W, CHUNK)
    zeros_in = jnp.zeros((N_ACC, D_IN), jnp.float32)
    zeros_d = jnp.zeros((N_ACC, D), jnp.float32)

    agg1 = _get_sc_scatter(D_IN)(x, zeros_in, src_pad, dst_pad)
    h1 = _tc_l1(agg1, x, W_rel1, W_root1, b1.reshape(1, D))
    agg2 = _get_sc_scatter(D)(h1, zeros_d, src_pad, dst_pad)
    out = _tc_post(agg2, h1, batch.astype(jnp.int32).reshape(N, 1),
                   W_rel2, W_root2, b2.reshape(1, D),
                   W_ih.T, W_hh.T, b_ih.reshape(1, 4 * D),
                   b_hh.reshape(1, 4 * D), W_dense, b_dense.reshape(1, D),
                   W_out, b_out.reshape(1, 1))
    return out.reshape(G)


# speedup vs baseline: 1.0562x; 1.0562x over previous
import functools

import jax
import jax.numpy as jnp
from jax import lax
from jax.experimental import pallas as pl
from jax.experimental.pallas import tpu as pltpu
from jax.experimental.pallas import tpu_sc as plsc

N = 10000
E = 320000
D_IN = 128
D = 64
G = 64
STEPS = 3

NC = 2
NS = 16
NW = NC * NS
CHUNK = 128
CHUNKS_PER_W = 80
HALF_CHUNKS = CHUNKS_PER_W // 2
NPAIR = HALF_CHUNKS // 2
E_PAD = NW * CHUNK * CHUNKS_PER_W
EDGES_PER_W = CHUNK * CHUNKS_PER_W
N_ACC = 10112
INIT_ROWS = N_ACC // NS
OUT_ROWS = N_ACC // NS

BLK = 1000
NBLK = N // BLK


def _dot(a, b):
    return jnp.dot(a, b, preferred_element_type=jnp.float32)


def _dot_hi(a, b):
    return jnp.dot(a, b, preferred_element_type=jnp.float32,
                   precision=lax.Precision.HIGHEST)


def _dot_t_hi(a, b):
    return lax.dot_general(a, b, (((0,), (0,)), ((), ())),
                           preferred_element_type=jnp.float32,
                           precision=lax.Precision.HIGHEST)


def _make_sc_body(width):
    def body(tab_hbm, zeros_hbm, src_hbm, dst_hbm, out_hbm,
             acc, src_v, dst_v, rows_v, sem):
        c = lax.axis_index("c")
        s = lax.axis_index("s")
        pltpu.sync_copy(zeros_hbm.at[pl.ds(s * INIT_ROWS, INIT_ROWS)],
                        acc.at[pl.ds(s * INIT_ROWS, INIT_ROWS)])
        plsc.subcore_barrier()

        base = (c * NS + s) * CHUNKS_PER_W

        def step(k, carry):
            pltpu.sync_copy(src_hbm.at[base + k], src_v)
            pltpu.sync_copy(dst_hbm.at[base + k], dst_v)
            pltpu.async_copy(tab_hbm.at[src_v], rows_v, sem).wait()
            pltpu.sync_copy(rows_v, acc.at[dst_v], add=True)
            return carry

        lax.fori_loop(0, CHUNKS_PER_W, step, 0)
        plsc.subcore_barrier()
        pltpu.sync_copy(acc.at[pl.ds(s * OUT_ROWS, OUT_ROWS)],
                        out_hbm.at[c].at[pl.ds(s * OUT_ROWS, OUT_ROWS)])

    return body


@functools.cache
def _get_sc_scatter(width):
    return pl.kernel(
        _make_sc_body(width),
        out_type=jax.ShapeDtypeStruct((NC, N_ACC, width), jnp.float32),
        mesh=plsc.VectorSubcoreMesh(core_axis_name="c", subcore_axis_name="s",
                                    num_cores=NC, num_subcores=NS),
        scratch_types=[
            pltpu.VMEM_SHARED((N_ACC, width), jnp.float32),
            pltpu.VMEM((CHUNK,), jnp.int32),
            pltpu.VMEM((CHUNK,), jnp.int32),
            pltpu.VMEM((CHUNK, width), jnp.float32),
            pltpu.SemaphoreType.DMA,
        ],
        compiler_params=pltpu.CompilerParams(use_tc_tiling_on_sc=False),
    )


def _sc_spmem_body(tab_hbm, zeros_hbm, src_hbm, dst_hbm, out_hbm,
                   tab_s, acc, src_v, dst_v, rows_v, sem):
    c = lax.axis_index("c")
    s = lax.axis_index("s")
    wid = c * NS + s
    pltpu.sync_copy(tab_hbm.at[pl.ds(s * INIT_ROWS, INIT_ROWS)],
                    tab_s.at[pl.ds(s * INIT_ROWS, INIT_ROWS)])
    pltpu.sync_copy(zeros_hbm.at[pl.ds(s * INIT_ROWS, INIT_ROWS)],
                    acc.at[pl.ds(s * INIT_ROWS, INIT_ROWS)])
    pltpu.sync_copy(src_hbm.at[pl.ds(wid * CHUNKS_PER_W, CHUNKS_PER_W)],
                    src_v)
    pltpu.sync_copy(dst_hbm.at[pl.ds(wid * CHUNKS_PER_W, CHUNKS_PER_W)],
                    dst_v)
    plsc.subcore_barrier()

    def step(k, carry):
        pltpu.async_copy(tab_s.at[src_v.at[k]], rows_v, sem).wait()
        pltpu.sync_copy(rows_v, acc.at[dst_v.at[k]], add=True)
        return carry

    lax.fori_loop(0, CHUNKS_PER_W, step, 0)
    plsc.subcore_barrier()
    pltpu.sync_copy(acc.at[pl.ds(s * OUT_ROWS, OUT_ROWS)],
                    out_hbm.at[c].at[pl.ds(s * OUT_ROWS, OUT_ROWS)])


@functools.cache
def _get_sc_scatter_spmem():
    return pl.kernel(
        _sc_spmem_body,
        out_type=jax.ShapeDtypeStruct((NC, N_ACC, D), jnp.float32),
        mesh=plsc.VectorSubcoreMesh(core_axis_name="c", subcore_axis_name="s",
                                    num_cores=NC, num_subcores=NS),
        scratch_types=[
            pltpu.VMEM_SHARED((N_ACC, D), jnp.float32),
            pltpu.VMEM_SHARED((N_ACC, D), jnp.float32),
            pltpu.VMEM((CHUNKS_PER_W, CHUNK), jnp.int32),
            pltpu.VMEM((CHUNKS_PER_W, CHUNK), jnp.int32),
            pltpu.VMEM((CHUNK, D), jnp.float32),
            pltpu.SemaphoreType.DMA,
        ],
        compiler_params=pltpu.CompilerParams(use_tc_tiling_on_sc=False),
    )


def _tc_l1_body(agg_ref, x_ref, wr_ref, wo_ref, b_ref, h_ref):
    aggs = agg_ref[0] + agg_ref[1]
    h_ref[...] = jnp.maximum(
        _dot(aggs, wr_ref[...]) + _dot(x_ref[...], wo_ref[...]) + b_ref[...],
        0.0)


_tc_l1 = pl.pallas_call(
    _tc_l1_body,
    grid=(NBLK,),
    in_specs=[
        pl.BlockSpec((2, BLK, D_IN), lambda i: (0, i, 0)),
        pl.BlockSpec((BLK, D_IN), lambda i: (i, 0)),
        pl.BlockSpec((D_IN, D), lambda i: (0, 0)),
        pl.BlockSpec((D_IN, D), lambda i: (0, 0)),
        pl.BlockSpec((1, D), lambda i: (0, 0)),
    ],
    out_specs=pl.BlockSpec((BLK, D), lambda i: (i, 0)),
    out_shape=jax.ShapeDtypeStruct((N, D), jnp.float32),
)


def _tc_post_body(agg_ref, h1_ref, batch_ref, wr_ref, wo2_ref, b2_ref,
                  wih_ref, whh_ref, bih_ref, bhh_ref, wd_ref, bd_ref,
                  wo_ref, bo_ref, out_ref, h_s):
    seg = lax.broadcasted_iota(jnp.int32, (1, G), 1)
    bg = bih_ref[...] + bhh_ref[...]

    def init_blk(i, carry):
        blk = pl.ds(i * BLK, BLK)
        aggs = agg_ref[0, blk, :] + agg_ref[1, blk, :]
        h_s[blk, :] = jnp.maximum(
            _dot(aggs, wr_ref[...]) + _dot(h1_ref[blk, :], wo2_ref[...])
            + b2_ref[...], 0.0)
        return carry

    lax.fori_loop(0, NBLK, init_blk, 0)

    q_star = jnp.zeros((G, 2 * D), jnp.float32)
    hh = jnp.zeros((G, D), jnp.float32)
    cc = jnp.zeros((G, D), jnp.float32)
    for _ in range(STEPS):
        gates = _dot(q_star, wih_ref[...]) + _dot(hh, whh_ref[...]) + bg
        gi = jax.nn.sigmoid(gates[:, 0:D])
        gf = jax.nn.sigmoid(gates[:, D:2 * D])
        gg = jnp.tanh(gates[:, 2 * D:3 * D])
        go = jax.nn.sigmoid(gates[:, 3 * D:4 * D])
        cc = gf * cc + gi * gg
        hh = go * jnp.tanh(cc)

        def pass_a(i, emax_c):
            blk = pl.ds(i * BLK, BLK)
            mb = batch_ref[blk, :] == seg
            qb = _dot_hi(mb.astype(jnp.float32), hh)
            e = jnp.sum(h_s[blk, :] * qb, axis=1, keepdims=True)
            blkmax = jnp.max(jnp.where(mb, e, -jnp.inf), axis=0,
                             keepdims=True)
            return jnp.maximum(emax_c, blkmax)

        emax = lax.fori_loop(0, NBLK, pass_a,
                             jnp.full((1, G), -jnp.inf, jnp.float32))

        def pass_b(i, carry):
            den_c, run_c = carry
            blk = pl.ds(i * BLK, BLK)
            mb = batch_ref[blk, :] == seg
            m = mb.astype(jnp.float32)
            hb = h_s[blk, :]
            qb = _dot_hi(m, hh)
            e = jnp.sum(hb * qb, axis=1, keepdims=True)
            emaxb = jnp.sum(jnp.where(mb, emax, 0.0), axis=1, keepdims=True)
            eexp = jnp.exp(e - emaxb)
            den_c = den_c + _dot_t_hi(m, eexp)
            run_c = run_c + _dot_t_hi(m, eexp * hb)
            return den_c, run_c

        den, run = lax.fori_loop(
            0, NBLK, pass_b,
            (jnp.zeros((G, 1), jnp.float32), jnp.zeros((G, D), jnp.float32)))
        r = run / jnp.where(den > 0.0, den, 1.0)
        q_star = jnp.concatenate([hh, r], axis=1)

    o = jnp.maximum(_dot(q_star, wd_ref[...]) + bd_ref[...], 0.0)
    out_ref[...] = _dot(o, wo_ref[...]) + bo_ref[...]


_tc_post = pl.pallas_call(
    _tc_post_body,
    out_shape=jax.ShapeDtypeStruct((G, 1), jnp.float32),
    scratch_shapes=[pltpu.VMEM((N, D), jnp.float32)],
)


def kernel(x, edge_index, batch, W_rel1, W_root1, b1, W_rel2, W_root2, b2,
           W_ih, W_hh, b_ih, b_hh, W_dense, b_dense, W_out, b_out):
    src = edge_index[0].astype(jnp.int32)
    dst = edge_index[1].astype(jnp.int32)
    pad = E_PAD - E
    src_pad = jnp.concatenate([src, jnp.zeros((pad,), jnp.int32)]).reshape(
        NW * CHUNKS_PER_W, CHUNK)
    dst_pad = jnp.concatenate([dst, jnp.full((pad,), N, jnp.int32)]).reshape(
        NW * CHUNKS_PER_W, CHUNK)
    zeros_in = jnp.zeros((N_ACC, D_IN), jnp.float32)
    zeros_d = jnp.zeros((N_ACC, D), jnp.float32)

    agg1 = _get_sc_scatter(D_IN)(x, zeros_in, src_pad, dst_pad)
    h1 = _tc_l1(agg1, x, W_rel1, W_root1, b1.reshape(1, D))
    h1_pad = jnp.pad(h1, ((0, N_ACC - N), (0, 0)))
    agg2 = _get_sc_scatter_spmem()(h1_pad, zeros_d, src_pad, dst_pad)
    out = _tc_post(agg2, h1, batch.astype(jnp.int32).reshape(N, 1),
                   W_rel2, W_root2, b2.reshape(1, D),
                   W_ih.T, W_hh.T, b_ih.reshape(1, 4 * D),
                   b_hh.reshape(1, 4 * D), W_dense, b_dense.reshape(1, D),
                   W_out, b_out.reshape(1, 1))
    return out.reshape(G)


# speedup vs baseline: 1.7083x; 1.6174x over previous
import functools

import jax
import jax.numpy as jnp
from jax import lax
from jax.experimental import pallas as pl
from jax.experimental.pallas import tpu as pltpu
from jax.experimental.pallas import tpu_sc as plsc

N = 10000
E = 320000
D_IN = 128
D = 64
G = 64
STEPS = 3

NC = 2
NS = 16
NW = NC * NS
CHUNK = 128
CHUNKS_PER_W = 80
HALF_CHUNKS = CHUNKS_PER_W // 2
NPAIR = HALF_CHUNKS // 2
E_PAD = NW * CHUNK * CHUNKS_PER_W
EDGES_PER_W = CHUNK * CHUNKS_PER_W
N_ACC = 10112
INIT_ROWS = N_ACC // NS
OUT_ROWS = N_ACC // NS

BLK = 1000
NBLK = N // BLK


def _dot(a, b):
    return jnp.dot(a, b, preferred_element_type=jnp.float32)


def _dot_hi(a, b):
    return jnp.dot(a, b, preferred_element_type=jnp.float32,
                   precision=lax.Precision.HIGHEST)


def _dot_t_hi(a, b):
    return lax.dot_general(a, b, (((0,), (0,)), ((), ())),
                           preferred_element_type=jnp.float32,
                           precision=lax.Precision.HIGHEST)


def _make_sc_spmem_body(nhalves):
    def body(tab_hbm, zeros_hbm, src_hbm, dst_hbm, out_hbm,
             tab_s, acc, src_v, dst_v, rows_v, sem):
        c = lax.axis_index("c")
        s = lax.axis_index("s")
        wid = c * NS + s
        slab = pl.ds(s * INIT_ROWS, INIT_ROWS)
        pltpu.sync_copy(src_hbm.at[pl.ds(wid * CHUNKS_PER_W, CHUNKS_PER_W)],
                        src_v)
        pltpu.sync_copy(dst_hbm.at[pl.ds(wid * CHUNKS_PER_W, CHUNKS_PER_W)],
                        dst_v)

        for h in range(nhalves):
            pltpu.sync_copy(tab_hbm.at[h].at[slab], tab_s.at[slab])
            pltpu.sync_copy(zeros_hbm.at[slab], acc.at[slab])
            plsc.subcore_barrier()

            def step(k, carry):
                pltpu.async_copy(tab_s.at[src_v.at[k]], rows_v, sem).wait()
                pltpu.sync_copy(rows_v, acc.at[dst_v.at[k]], add=True)
                return carry

            lax.fori_loop(0, CHUNKS_PER_W, step, 0)
            plsc.subcore_barrier()
            pltpu.sync_copy(acc.at[pl.ds(s * OUT_ROWS, OUT_ROWS)],
                            out_hbm.at[h].at[c].at[pl.ds(s * OUT_ROWS,
                                                         OUT_ROWS)])

    return body


@functools.cache
def _get_sc_scatter_spmem(nhalves):
    return pl.kernel(
        _make_sc_spmem_body(nhalves),
        out_type=jax.ShapeDtypeStruct((nhalves, NC, N_ACC, D), jnp.float32),
        mesh=plsc.VectorSubcoreMesh(core_axis_name="c", subcore_axis_name="s",
                                    num_cores=NC, num_subcores=NS),
        scratch_types=[
            pltpu.VMEM_SHARED((N_ACC, D), jnp.float32),
            pltpu.VMEM_SHARED((N_ACC, D), jnp.float32),
            pltpu.VMEM((CHUNKS_PER_W, CHUNK), jnp.int32),
            pltpu.VMEM((CHUNKS_PER_W, CHUNK), jnp.int32),
            pltpu.VMEM((CHUNK, D), jnp.float32),
            pltpu.SemaphoreType.DMA,
        ],
        compiler_params=pltpu.CompilerParams(use_tc_tiling_on_sc=False),
    )


def _tc_l1_body(agg_ref, x_ref, wra_ref, wrb_ref, wo_ref, b_ref, h_ref):
    aggs_a = agg_ref[0, 0] + agg_ref[0, 1]
    aggs_b = agg_ref[1, 0] + agg_ref[1, 1]
    h_ref[...] = jnp.maximum(
        _dot(aggs_a, wra_ref[...]) + _dot(aggs_b, wrb_ref[...])
        + _dot(x_ref[...], wo_ref[...]) + b_ref[...], 0.0)


_tc_l1 = pl.pallas_call(
    _tc_l1_body,
    grid=(NBLK,),
    in_specs=[
        pl.BlockSpec((2, 2, BLK, D), lambda i: (0, 0, i, 0)),
        pl.BlockSpec((BLK, D_IN), lambda i: (i, 0)),
        pl.BlockSpec((D, D), lambda i: (0, 0)),
        pl.BlockSpec((D, D), lambda i: (0, 0)),
        pl.BlockSpec((D_IN, D), lambda i: (0, 0)),
        pl.BlockSpec((1, D), lambda i: (0, 0)),
    ],
    out_specs=pl.BlockSpec((BLK, D), lambda i: (i, 0)),
    out_shape=jax.ShapeDtypeStruct((N, D), jnp.float32),
)


def _tc_post_body(agg_ref, h1_ref, batch_ref, wr_ref, wo2_ref, b2_ref,
                  wih_ref, whh_ref, bih_ref, bhh_ref, wd_ref, bd_ref,
                  wo_ref, bo_ref, out_ref, h_s):
    seg = lax.broadcasted_iota(jnp.int32, (1, G), 1)
    bg = bih_ref[...] + bhh_ref[...]

    def init_blk(i, carry):
        blk = pl.ds(i * BLK, BLK)
        aggs = agg_ref[0, blk, :] + agg_ref[1, blk, :]
        h_s[blk, :] = jnp.maximum(
            _dot(aggs, wr_ref[...]) + _dot(h1_ref[blk, :], wo2_ref[...])
            + b2_ref[...], 0.0)
        return carry

    lax.fori_loop(0, NBLK, init_blk, 0)

    q_star = jnp.zeros((G, 2 * D), jnp.float32)
    hh = jnp.zeros((G, D), jnp.float32)
    cc = jnp.zeros((G, D), jnp.float32)
    for _ in range(STEPS):
        gates = _dot(q_star, wih_ref[...]) + _dot(hh, whh_ref[...]) + bg
        gi = jax.nn.sigmoid(gates[:, 0:D])
        gf = jax.nn.sigmoid(gates[:, D:2 * D])
        gg = jnp.tanh(gates[:, 2 * D:3 * D])
        go = jax.nn.sigmoid(gates[:, 3 * D:4 * D])
        cc = gf * cc + gi * gg
        hh = go * jnp.tanh(cc)

        def pass_a(i, emax_c):
            blk = pl.ds(i * BLK, BLK)
            mb = batch_ref[blk, :] == seg
            qb = _dot_hi(mb.astype(jnp.float32), hh)
            e = jnp.sum(h_s[blk, :] * qb, axis=1, keepdims=True)
            blkmax = jnp.max(jnp.where(mb, e, -jnp.inf), axis=0,
                             keepdims=True)
            return jnp.maximum(emax_c, blkmax)

        emax = lax.fori_loop(0, NBLK, pass_a,
                             jnp.full((1, G), -jnp.inf, jnp.float32))

        def pass_b(i, carry):
            den_c, run_c = carry
            blk = pl.ds(i * BLK, BLK)
            mb = batch_ref[blk, :] == seg
            m = mb.astype(jnp.float32)
            hb = h_s[blk, :]
            qb = _dot_hi(m, hh)
            e = jnp.sum(hb * qb, axis=1, keepdims=True)
            emaxb = jnp.sum(jnp.where(mb, emax, 0.0), axis=1, keepdims=True)
            eexp = jnp.exp(e - emaxb)
            den_c = den_c + _dot_t_hi(m, eexp)
            run_c = run_c + _dot_t_hi(m, eexp * hb)
            return den_c, run_c

        den, run = lax.fori_loop(
            0, NBLK, pass_b,
            (jnp.zeros((G, 1), jnp.float32), jnp.zeros((G, D), jnp.float32)))
        r = run / jnp.where(den > 0.0, den, 1.0)
        q_star = jnp.concatenate([hh, r], axis=1)

    o = jnp.maximum(_dot(q_star, wd_ref[...]) + bd_ref[...], 0.0)
    out_ref[...] = _dot(o, wo_ref[...]) + bo_ref[...]


_tc_post = pl.pallas_call(
    _tc_post_body,
    out_shape=jax.ShapeDtypeStruct((G, 1), jnp.float32),
    scratch_shapes=[pltpu.VMEM((N, D), jnp.float32)],
)


def kernel(x, edge_index, batch, W_rel1, W_root1, b1, W_rel2, W_root2, b2,
           W_ih, W_hh, b_ih, b_hh, W_dense, b_dense, W_out, b_out):
    src = edge_index[0].astype(jnp.int32)
    dst = edge_index[1].astype(jnp.int32)
    pad = E_PAD - E
    src_pad = jnp.concatenate([src, jnp.zeros((pad,), jnp.int32)]).reshape(
        NW * CHUNKS_PER_W, CHUNK)
    dst_pad = jnp.concatenate([dst, jnp.full((pad,), N, jnp.int32)]).reshape(
        NW * CHUNKS_PER_W, CHUNK)
    zeros_d = jnp.zeros((N_ACC, D), jnp.float32)
    xtab = jnp.stack([jnp.pad(x[:, :D], ((0, N_ACC - N), (0, 0))),
                      jnp.pad(x[:, D:], ((0, N_ACC - N), (0, 0)))])

    agg1 = _get_sc_scatter_spmem(2)(xtab, zeros_d, src_pad, dst_pad)
    h1 = _tc_l1(agg1, x, W_rel1[:D], W_rel1[D:], W_root1, b1.reshape(1, D))
    h1_pad = jnp.pad(h1, ((0, N_ACC - N), (0, 0)))
    agg2 = _get_sc_scatter_spmem(1)(h1_pad.reshape(1, N_ACC, D), zeros_d,
                                    src_pad, dst_pad).reshape(NC, N_ACC, D)
    out = _tc_post(agg2, h1, batch.astype(jnp.int32).reshape(N, 1),
                   W_rel2, W_root2, b2.reshape(1, D),
                   W_ih.T, W_hh.T, b_ih.reshape(1, 4 * D),
                   b_hh.reshape(1, 4 * D), W_dense, b_dense.reshape(1, D),
                   W_out, b_out.reshape(1, 1))
    return out.reshape(G)


# speedup vs baseline: 2.0479x; 1.1988x over previous
import functools

import jax
import jax.numpy as jnp
from jax import lax
from jax.experimental import pallas as pl
from jax.experimental.pallas import tpu as pltpu
from jax.experimental.pallas import tpu_sc as plsc

N = 10000
E = 320000
D_IN = 128
D = 64
G = 64
STEPS = 3

NC = 2
NS = 16
NW = NC * NS
CHUNK = 128
CHUNKS_PER_W = 80
HALF_CHUNKS = CHUNKS_PER_W // 2
NPAIR = HALF_CHUNKS // 2
E_PAD = NW * CHUNK * CHUNKS_PER_W
EDGES_PER_W = CHUNK * CHUNKS_PER_W
N_ACC = 10112
INIT_ROWS = N_ACC // NS
OUT_ROWS = N_ACC // NS

BLK = 1000
NBLK = N // BLK


def _dot(a, b):
    return jnp.dot(a, b, preferred_element_type=jnp.float32)


def _dot_hi(a, b):
    return jnp.dot(a, b, preferred_element_type=jnp.float32,
                   precision=lax.Precision.HIGHEST)


def _dot_t_hi(a, b):
    return lax.dot_general(a, b, (((0,), (0,)), ((), ())),
                           preferred_element_type=jnp.float32,
                           precision=lax.Precision.HIGHEST)


def _make_sc_spmem_body(nhalves):
    def body(tab_hbm, zeros_hbm, src_hbm, dst_hbm, out_hbm,
             tab_s, acc, src_v, dst_v, rows_v, rows_w, sem, sem2):
        c = lax.axis_index("c")
        s = lax.axis_index("s")
        wid = c * NS + s
        slab = pl.ds(s * INIT_ROWS, INIT_ROWS)
        pltpu.sync_copy(src_hbm.at[pl.ds(wid * CHUNKS_PER_W, CHUNKS_PER_W)],
                        src_v)
        pltpu.sync_copy(dst_hbm.at[pl.ds(wid * CHUNKS_PER_W, CHUNKS_PER_W)],
                        dst_v)

        for h in range(nhalves):
            pltpu.sync_copy(tab_hbm.at[h].at[slab], tab_s.at[slab])
            pltpu.sync_copy(zeros_hbm.at[slab], acc.at[slab])
            plsc.subcore_barrier()

            pltpu.async_copy(tab_s.at[src_v.at[0]], rows_v, sem)

            def pair(kk, carry):
                a = 2 * kk
                pltpu.make_async_copy(tab_s.at[src_v.at[a]], rows_v,
                                      sem).wait()
                pltpu.async_copy(tab_s.at[src_v.at[a + 1]], rows_w, sem2)
                pltpu.sync_copy(rows_v, acc.at[dst_v.at[a]], add=True)
                pltpu.make_async_copy(tab_s.at[src_v.at[a + 1]], rows_w,
                                      sem2).wait()
                nxt = jnp.minimum(a + 2, CHUNKS_PER_W - 1)
                pltpu.async_copy(tab_s.at[src_v.at[nxt]], rows_v, sem)
                pltpu.sync_copy(rows_w, acc.at[dst_v.at[a + 1]], add=True)
                return carry

            lax.fori_loop(0, CHUNKS_PER_W // 2, pair, 0)
            pltpu.make_async_copy(tab_s.at[src_v.at[0]], rows_v, sem).wait()
            plsc.subcore_barrier()
            pltpu.sync_copy(acc.at[pl.ds(s * OUT_ROWS, OUT_ROWS)],
                            out_hbm.at[h].at[c].at[pl.ds(s * OUT_ROWS,
                                                         OUT_ROWS)])

    return body


@functools.cache
def _get_sc_scatter_spmem(nhalves):
    return pl.kernel(
        _make_sc_spmem_body(nhalves),
        out_type=jax.ShapeDtypeStruct((nhalves, NC, N_ACC, D), jnp.float32),
        mesh=plsc.VectorSubcoreMesh(core_axis_name="c", subcore_axis_name="s",
                                    num_cores=NC, num_subcores=NS),
        scratch_types=[
            pltpu.VMEM_SHARED((N_ACC, D), jnp.float32),
            pltpu.VMEM_SHARED((N_ACC, D), jnp.float32),
            pltpu.VMEM((CHUNKS_PER_W, CHUNK), jnp.int32),
            pltpu.VMEM((CHUNKS_PER_W, CHUNK), jnp.int32),
            pltpu.VMEM((CHUNK, D), jnp.float32),
            pltpu.VMEM((CHUNK, D), jnp.float32),
            pltpu.SemaphoreType.DMA,
            pltpu.SemaphoreType.DMA,
        ],
        compiler_params=pltpu.CompilerParams(use_tc_tiling_on_sc=False),
    )


def _tc_l1_body(agg_ref, x_ref, wra_ref, wrb_ref, wo_ref, b_ref, h_ref):
    aggs_a = agg_ref[0, 0] + agg_ref[0, 1]
    aggs_b = agg_ref[1, 0] + agg_ref[1, 1]
    h_ref[...] = jnp.maximum(
        _dot(aggs_a, wra_ref[...]) + _dot(aggs_b, wrb_ref[...])
        + _dot(x_ref[...], wo_ref[...]) + b_ref[...], 0.0)


_tc_l1 = pl.pallas_call(
    _tc_l1_body,
    grid=(NBLK,),
    in_specs=[
        pl.BlockSpec((2, 2, BLK, D), lambda i: (0, 0, i, 0)),
        pl.BlockSpec((BLK, D_IN), lambda i: (i, 0)),
        pl.BlockSpec((D, D), lambda i: (0, 0)),
        pl.BlockSpec((D, D), lambda i: (0, 0)),
        pl.BlockSpec((D_IN, D), lambda i: (0, 0)),
        pl.BlockSpec((1, D), lambda i: (0, 0)),
    ],
    out_specs=pl.BlockSpec((BLK, D), lambda i: (i, 0)),
    out_shape=jax.ShapeDtypeStruct((N, D), jnp.float32),
)


def _tc_post_body(agg_ref, h1_ref, batch_ref, wr_ref, wo2_ref, b2_ref,
                  wih_ref, whh_ref, bih_ref, bhh_ref, wd_ref, bd_ref,
                  wo_ref, bo_ref, out_ref, h_s):
    seg = lax.broadcasted_iota(jnp.int32, (1, G), 1)
    bg = bih_ref[...] + bhh_ref[...]

    def init_blk(i, carry):
        blk = pl.ds(i * BLK, BLK)
        aggs = agg_ref[0, blk, :] + agg_ref[1, blk, :]
        h_s[blk, :] = jnp.maximum(
            _dot(aggs, wr_ref[...]) + _dot(h1_ref[blk, :], wo2_ref[...])
            + b2_ref[...], 0.0)
        return carry

    lax.fori_loop(0, NBLK, init_blk, 0)

    q_star = jnp.zeros((G, 2 * D), jnp.float32)
    hh = jnp.zeros((G, D), jnp.float32)
    cc = jnp.zeros((G, D), jnp.float32)
    for _ in range(STEPS):
        gates = _dot(q_star, wih_ref[...]) + _dot(hh, whh_ref[...]) + bg
        gi = jax.nn.sigmoid(gates[:, 0:D])
        gf = jax.nn.sigmoid(gates[:, D:2 * D])
        gg = jnp.tanh(gates[:, 2 * D:3 * D])
        go = jax.nn.sigmoid(gates[:, 3 * D:4 * D])
        cc = gf * cc + gi * gg
        hh = go * jnp.tanh(cc)

        def pass_a(i, emax_c):
            blk = pl.ds(i * BLK, BLK)
            mb = batch_ref[blk, :] == seg
            qb = _dot_hi(mb.astype(jnp.float32), hh)
            e = jnp.sum(h_s[blk, :] * qb, axis=1, keepdims=True)
            blkmax = jnp.max(jnp.where(mb, e, -jnp.inf), axis=0,
                             keepdims=True)
            return jnp.maximum(emax_c, blkmax)

        emax = lax.fori_loop(0, NBLK, pass_a,
                             jnp.full((1, G), -jnp.inf, jnp.float32))

        def pass_b(i, carry):
            den_c, run_c = carry
            blk = pl.ds(i * BLK, BLK)
            mb = batch_ref[blk, :] == seg
            m = mb.astype(jnp.float32)
            hb = h_s[blk, :]
            qb = _dot_hi(m, hh)
            e = jnp.sum(hb * qb, axis=1, keepdims=True)
            emaxb = jnp.sum(jnp.where(mb, emax, 0.0), axis=1, keepdims=True)
            eexp = jnp.exp(e - emaxb)
            den_c = den_c + _dot_t_hi(m, eexp)
            run_c = run_c + _dot_t_hi(m, eexp * hb)
            return den_c, run_c

        den, run = lax.fori_loop(
            0, NBLK, pass_b,
            (jnp.zeros((G, 1), jnp.float32), jnp.zeros((G, D), jnp.float32)))
        r = run / jnp.where(den > 0.0, den, 1.0)
        q_star = jnp.concatenate([hh, r], axis=1)

    o = jnp.maximum(_dot(q_star, wd_ref[...]) + bd_ref[...], 0.0)
    out_ref[...] = _dot(o, wo_ref[...]) + bo_ref[...]


_tc_post = pl.pallas_call(
    _tc_post_body,
    out_shape=jax.ShapeDtypeStruct((G, 1), jnp.float32),
    scratch_shapes=[pltpu.VMEM((N, D), jnp.float32)],
)


def kernel(x, edge_index, batch, W_rel1, W_root1, b1, W_rel2, W_root2, b2,
           W_ih, W_hh, b_ih, b_hh, W_dense, b_dense, W_out, b_out):
    src = edge_index[0].astype(jnp.int32)
    dst = edge_index[1].astype(jnp.int32)
    pad = E_PAD - E
    src_pad = jnp.concatenate([src, jnp.zeros((pad,), jnp.int32)]).reshape(
        NW * CHUNKS_PER_W, CHUNK)
    dst_pad = jnp.concatenate([dst, jnp.full((pad,), N, jnp.int32)]).reshape(
        NW * CHUNKS_PER_W, CHUNK)
    zeros_d = jnp.zeros((N_ACC, D), jnp.float32)
    xtab = jnp.stack([jnp.pad(x[:, :D], ((0, N_ACC - N), (0, 0))),
                      jnp.pad(x[:, D:], ((0, N_ACC - N), (0, 0)))])

    agg1 = _get_sc_scatter_spmem(2)(xtab, zeros_d, src_pad, dst_pad)
    h1 = _tc_l1(agg1, x, W_rel1[:D], W_rel1[D:], W_root1, b1.reshape(1, D))
    h1_pad = jnp.pad(h1, ((0, N_ACC - N), (0, 0)))
    agg2 = _get_sc_scatter_spmem(1)(h1_pad.reshape(1, N_ACC, D), zeros_d,
                                    src_pad, dst_pad).reshape(NC, N_ACC, D)
    out = _tc_post(agg2, h1, batch.astype(jnp.int32).reshape(N, 1),
                   W_rel2, W_root2, b2.reshape(1, D),
                   W_ih.T, W_hh.T, b_ih.reshape(1, 4 * D),
                   b_hh.reshape(1, 4 * D), W_dense, b_dense.reshape(1, D),
                   W_out, b_out.reshape(1, 1))
    return out.reshape(G)


# speedup vs baseline: 2.2207x; 1.0844x over previous
import functools

import jax
import jax.numpy as jnp
from jax import lax
from jax.experimental import pallas as pl
from jax.experimental.pallas import tpu as pltpu
from jax.experimental.pallas import tpu_sc as plsc

N = 10000
E = 320000
D_IN = 128
D = 64
G = 64
STEPS = 3

NC = 2
NS = 16
NW = NC * NS
CHUNK = 128
CHUNKS_PER_W = 80
HALF_CHUNKS = CHUNKS_PER_W // 2
NPAIR = HALF_CHUNKS // 2
E_PAD = NW * CHUNK * CHUNKS_PER_W
EDGES_PER_W = CHUNK * CHUNKS_PER_W
N_ACC = 10112
INIT_ROWS = N_ACC // NS
OUT_ROWS = N_ACC // NS

BLK = 1000
NBLK = N // BLK


def _dot(a, b):
    return jnp.dot(a, b, preferred_element_type=jnp.float32)


def _dot_hi(a, b):
    return jnp.dot(a, b, preferred_element_type=jnp.float32,
                   precision=lax.Precision.HIGHEST)


def _dot_t_hi(a, b):
    return lax.dot_general(a, b, (((0,), (0,)), ((), ())),
                           preferred_element_type=jnp.float32,
                           precision=lax.Precision.HIGHEST)


def _make_sc_spmem_body(nhalves):
    def body(tab_hbm, zeros_hbm, src_hbm, dst_hbm, out_hbm,
             tab_s, acc, src_v, dst_v, rows_v, rows_w, sem, sem2):
        c = lax.axis_index("c")
        s = lax.axis_index("s")
        wid = c * NS + s
        slab = pl.ds(s * INIT_ROWS, INIT_ROWS)
        pltpu.sync_copy(src_hbm.at[pl.ds(wid * CHUNKS_PER_W, CHUNKS_PER_W)],
                        src_v)
        pltpu.sync_copy(dst_hbm.at[pl.ds(wid * CHUNKS_PER_W, CHUNKS_PER_W)],
                        dst_v)

        for h in range(nhalves):
            pltpu.sync_copy(tab_hbm.at[h].at[slab], tab_s.at[slab])
            pltpu.sync_copy(zeros_hbm.at[slab], acc.at[slab])
            plsc.subcore_barrier()

            pltpu.async_copy(tab_s.at[src_v.at[0]], rows_v, sem)

            def pair(kk, carry):
                a = 2 * kk
                pltpu.make_async_copy(tab_s.at[src_v.at[a]], rows_v,
                                      sem).wait()
                pltpu.async_copy(tab_s.at[src_v.at[a + 1]], rows_w, sem2)
                pltpu.sync_copy(rows_v, acc.at[dst_v.at[a]], add=True)
                pltpu.make_async_copy(tab_s.at[src_v.at[a + 1]], rows_w,
                                      sem2).wait()
                nxt = jnp.minimum(a + 2, CHUNKS_PER_W - 1)
                pltpu.async_copy(tab_s.at[src_v.at[nxt]], rows_v, sem)
                pltpu.sync_copy(rows_w, acc.at[dst_v.at[a + 1]], add=True)
                return carry

            lax.fori_loop(0, CHUNKS_PER_W // 2, pair, 0)
            pltpu.make_async_copy(tab_s.at[src_v.at[0]], rows_v, sem).wait()
            plsc.subcore_barrier()
            pltpu.sync_copy(acc.at[pl.ds(s * OUT_ROWS, OUT_ROWS)],
                            out_hbm.at[h].at[c].at[pl.ds(s * OUT_ROWS,
                                                         OUT_ROWS)])

    return body


@functools.cache
def _get_sc_scatter_spmem(nhalves):
    return pl.kernel(
        _make_sc_spmem_body(nhalves),
        out_type=jax.ShapeDtypeStruct((nhalves, NC, N_ACC, D), jnp.float32),
        mesh=plsc.VectorSubcoreMesh(core_axis_name="c", subcore_axis_name="s",
                                    num_cores=NC, num_subcores=NS),
        scratch_types=[
            pltpu.VMEM_SHARED((N_ACC, D), jnp.float32),
            pltpu.VMEM_SHARED((N_ACC, D), jnp.float32),
            pltpu.VMEM((CHUNKS_PER_W, CHUNK), jnp.int32),
            pltpu.VMEM((CHUNKS_PER_W, CHUNK), jnp.int32),
            pltpu.VMEM((CHUNK, D), jnp.float32),
            pltpu.VMEM((CHUNK, D), jnp.float32),
            pltpu.SemaphoreType.DMA,
            pltpu.SemaphoreType.DMA,
        ],
        compiler_params=pltpu.CompilerParams(use_tc_tiling_on_sc=False),
    )


def _tc_l1_body(agg_ref, x_ref, wra_ref, wrb_ref, wo_ref, b_ref, h_ref):
    aggs_a = agg_ref[0, 0] + agg_ref[0, 1]
    aggs_b = agg_ref[1, 0] + agg_ref[1, 1]
    h_ref[...] = jnp.maximum(
        _dot(aggs_a, wra_ref[...]) + _dot(aggs_b, wrb_ref[...])
        + _dot(x_ref[...], wo_ref[...]) + b_ref[...], 0.0)


_tc_l1 = pl.pallas_call(
    _tc_l1_body,
    grid=(NBLK,),
    in_specs=[
        pl.BlockSpec((2, 2, BLK, D), lambda i: (0, 0, i, 0)),
        pl.BlockSpec((BLK, D_IN), lambda i: (i, 0)),
        pl.BlockSpec((D, D), lambda i: (0, 0)),
        pl.BlockSpec((D, D), lambda i: (0, 0)),
        pl.BlockSpec((D_IN, D), lambda i: (0, 0)),
        pl.BlockSpec((1, D), lambda i: (0, 0)),
    ],
    out_specs=pl.BlockSpec((BLK, D), lambda i: (i, 0)),
    out_shape=jax.ShapeDtypeStruct((N, D), jnp.float32),
)


def _tc_post_body(agg_ref, h1_ref, batch_ref, wr_ref, wo2_ref, b2_ref,
                  wih_ref, whh_ref, bih_ref, bhh_ref, wd_ref, bd_ref,
                  wo_ref, bo_ref, out_ref, h_s, e_s):
    seg = lax.broadcasted_iota(jnp.int32, (1, G), 1)
    bg = bih_ref[...] + bhh_ref[...]

    def init_blk(i, carry):
        blk = pl.ds(i * BLK, BLK)
        aggs = agg_ref[0, blk, :] + agg_ref[1, blk, :]
        h_s[blk, :] = jnp.maximum(
            _dot(aggs, wr_ref[...]) + _dot(h1_ref[blk, :], wo2_ref[...])
            + b2_ref[...], 0.0)
        return carry

    lax.fori_loop(0, NBLK, init_blk, 0)

    q_star = jnp.zeros((G, 2 * D), jnp.float32)
    hh = jnp.zeros((G, D), jnp.float32)
    cc = jnp.zeros((G, D), jnp.float32)
    for _ in range(STEPS):
        gates = _dot(q_star, wih_ref[...]) + _dot(hh, whh_ref[...]) + bg
        gi = jax.nn.sigmoid(gates[:, 0:D])
        gf = jax.nn.sigmoid(gates[:, D:2 * D])
        gg = jnp.tanh(gates[:, 2 * D:3 * D])
        go = jax.nn.sigmoid(gates[:, 3 * D:4 * D])
        cc = gf * cc + gi * gg
        hh = go * jnp.tanh(cc)

        def pass_a(i, emax_c):
            blk = pl.ds(i * BLK, BLK)
            mb = batch_ref[blk, :] == seg
            qb = _dot_hi(mb.astype(jnp.float32), hh)
            e = jnp.sum(h_s[blk, :] * qb, axis=1, keepdims=True)
            e_s[blk, :] = e
            blkmax = jnp.max(jnp.where(mb, e, -jnp.inf), axis=0,
                             keepdims=True)
            return jnp.maximum(emax_c, blkmax)

        emax = lax.fori_loop(0, NBLK, pass_a,
                             jnp.full((1, G), -jnp.inf, jnp.float32))

        def pass_b(i, carry):
            den_c, run_c = carry
            blk = pl.ds(i * BLK, BLK)
            mb = batch_ref[blk, :] == seg
            m = mb.astype(jnp.float32)
            hb = h_s[blk, :]
            e = e_s[blk, :]
            emaxb = jnp.sum(jnp.where(mb, emax, 0.0), axis=1, keepdims=True)
            eexp = jnp.exp(e - emaxb)
            den_c = den_c + _dot_t_hi(m, eexp)
            run_c = run_c + _dot_t_hi(m, eexp * hb)
            return den_c, run_c

        den, run = lax.fori_loop(
            0, NBLK, pass_b,
            (jnp.zeros((G, 1), jnp.float32), jnp.zeros((G, D), jnp.float32)))
        r = run / jnp.where(den > 0.0, den, 1.0)
        q_star = jnp.concatenate([hh, r], axis=1)

    o = jnp.maximum(_dot(q_star, wd_ref[...]) + bd_ref[...], 0.0)
    out_ref[...] = _dot(o, wo_ref[...]) + bo_ref[...]


_tc_post = pl.pallas_call(
    _tc_post_body,
    out_shape=jax.ShapeDtypeStruct((G, 1), jnp.float32),
    scratch_shapes=[pltpu.VMEM((N, D), jnp.float32),
                    pltpu.VMEM((N, 1), jnp.float32)],
)


def kernel(x, edge_index, batch, W_rel1, W_root1, b1, W_rel2, W_root2, b2,
           W_ih, W_hh, b_ih, b_hh, W_dense, b_dense, W_out, b_out):
    src = edge_index[0].astype(jnp.int32)
    dst = edge_index[1].astype(jnp.int32)
    pad = E_PAD - E
    src_pad = jnp.concatenate([src, jnp.zeros((pad,), jnp.int32)]).reshape(
        NW * CHUNKS_PER_W, CHUNK)
    dst_pad = jnp.concatenate([dst, jnp.full((pad,), N, jnp.int32)]).reshape(
        NW * CHUNKS_PER_W, CHUNK)
    zeros_d = jnp.zeros((N_ACC, D), jnp.float32)
    xtab = jnp.stack([jnp.pad(x[:, :D], ((0, N_ACC - N), (0, 0))),
                      jnp.pad(x[:, D:], ((0, N_ACC - N), (0, 0)))])

    agg1 = _get_sc_scatter_spmem(2)(xtab, zeros_d, src_pad, dst_pad)
    h1 = _tc_l1(agg1, x, W_rel1[:D], W_rel1[D:], W_root1, b1.reshape(1, D))
    h1_pad = jnp.pad(h1, ((0, N_ACC - N), (0, 0)))
    agg2 = _get_sc_scatter_spmem(1)(h1_pad.reshape(1, N_ACC, D), zeros_d,
                                    src_pad, dst_pad).reshape(NC, N_ACC, D)
    out = _tc_post(agg2, h1, batch.astype(jnp.int32).reshape(N, 1),
                   W_rel2, W_root2, b2.reshape(1, D),
                   W_ih.T, W_hh.T, b_ih.reshape(1, 4 * D),
                   b_hh.reshape(1, 4 * D), W_dense, b_dense.reshape(1, D),
                   W_out, b_out.reshape(1, 1))
    return out.reshape(G)


# speedup vs baseline: 2.2280x; 1.0033x over previous
import functools

import jax
import jax.numpy as jnp
from jax import lax
from jax.experimental import pallas as pl
from jax.experimental.pallas import tpu as pltpu
from jax.experimental.pallas import tpu_sc as plsc

N = 10000
E = 320000
D_IN = 128
D = 64
G = 64
STEPS = 3

NC = 2
NS = 16
NW = NC * NS
CHUNK = 128
CHUNKS_PER_W = 80
HALF_CHUNKS = CHUNKS_PER_W // 2
NPAIR = HALF_CHUNKS // 2
E_PAD = NW * CHUNK * CHUNKS_PER_W
EDGES_PER_W = CHUNK * CHUNKS_PER_W
N_ACC = 10112
INIT_ROWS = N_ACC // NS
OUT_ROWS = N_ACC // NS

BLK = 1000
NBLK = N // BLK


def _dot(a, b):
    return jnp.dot(a, b, preferred_element_type=jnp.float32)


def _dot_hi(a, b):
    return jnp.dot(a, b, preferred_element_type=jnp.float32,
                   precision=lax.Precision.HIGHEST)


def _dot_t_hi(a, b):
    return lax.dot_general(a, b, (((0,), (0,)), ((), ())),
                           preferred_element_type=jnp.float32,
                           precision=lax.Precision.HIGHEST)


def _make_sc_spmem_body(nhalves):
    def body(tab_hbm, zeros_hbm, src_hbm, dst_hbm, out_hbm,
             tab_s, acc, src_v, dst_v, rows_v, rows_w, sem, sem2):
        c = lax.axis_index("c")
        s = lax.axis_index("s")
        wid = c * NS + s
        slab = pl.ds(s * INIT_ROWS, INIT_ROWS)
        pltpu.sync_copy(src_hbm.at[pl.ds(wid * CHUNKS_PER_W, CHUNKS_PER_W)],
                        src_v)
        pltpu.sync_copy(dst_hbm.at[pl.ds(wid * CHUNKS_PER_W, CHUNKS_PER_W)],
                        dst_v)

        for h in range(nhalves):
            pltpu.sync_copy(tab_hbm.at[h].at[slab], tab_s.at[slab])
            pltpu.sync_copy(zeros_hbm.at[slab], acc.at[slab])
            plsc.subcore_barrier()

            pltpu.async_copy(tab_s.at[src_v.at[0]], rows_v, sem)

            def pair(kk, carry):
                a = 2 * kk
                pltpu.make_async_copy(tab_s.at[src_v.at[a]], rows_v,
                                      sem).wait()
                pltpu.async_copy(tab_s.at[src_v.at[a + 1]], rows_w, sem2)
                pltpu.sync_copy(rows_v, acc.at[dst_v.at[a]], add=True)
                pltpu.make_async_copy(tab_s.at[src_v.at[a + 1]], rows_w,
                                      sem2).wait()
                nxt = jnp.minimum(a + 2, CHUNKS_PER_W - 1)
                pltpu.async_copy(tab_s.at[src_v.at[nxt]], rows_v, sem)
                pltpu.sync_copy(rows_w, acc.at[dst_v.at[a + 1]], add=True)
                return carry

            lax.fori_loop(0, CHUNKS_PER_W // 2, pair, 0)
            pltpu.make_async_copy(tab_s.at[src_v.at[0]], rows_v, sem).wait()
            plsc.subcore_barrier()
            pltpu.sync_copy(acc.at[pl.ds(s * OUT_ROWS, OUT_ROWS)],
                            out_hbm.at[h].at[c].at[pl.ds(s * OUT_ROWS,
                                                         OUT_ROWS)])

    return body


@functools.cache
def _get_sc_scatter_spmem(nhalves):
    return pl.kernel(
        _make_sc_spmem_body(nhalves),
        out_type=jax.ShapeDtypeStruct((nhalves, NC, N_ACC, D), jnp.float32),
        mesh=plsc.VectorSubcoreMesh(core_axis_name="c", subcore_axis_name="s",
                                    num_cores=NC, num_subcores=NS),
        scratch_types=[
            pltpu.VMEM_SHARED((N_ACC, D), jnp.float32),
            pltpu.VMEM_SHARED((N_ACC, D), jnp.float32),
            pltpu.VMEM((CHUNKS_PER_W, CHUNK), jnp.int32),
            pltpu.VMEM((CHUNKS_PER_W, CHUNK), jnp.int32),
            pltpu.VMEM((CHUNK, D), jnp.float32),
            pltpu.VMEM((CHUNK, D), jnp.float32),
            pltpu.SemaphoreType.DMA,
            pltpu.SemaphoreType.DMA,
        ],
        compiler_params=pltpu.CompilerParams(use_tc_tiling_on_sc=False),
    )


def _tc_l1_body(agg_ref, x_ref, wra_ref, wrb_ref, wo_ref, b_ref, h_ref):
    aggs_a = agg_ref[0, 0] + agg_ref[0, 1]
    aggs_b = agg_ref[1, 0] + agg_ref[1, 1]
    h_ref[...] = jnp.maximum(
        _dot(aggs_a, wra_ref[...]) + _dot(aggs_b, wrb_ref[...])
        + _dot(x_ref[...], wo_ref[...]) + b_ref[...], 0.0)


_tc_l1 = pl.pallas_call(
    _tc_l1_body,
    grid=(NBLK,),
    in_specs=[
        pl.BlockSpec((2, 2, BLK, D), lambda i: (0, 0, i, 0)),
        pl.BlockSpec((BLK, D_IN), lambda i: (i, 0)),
        pl.BlockSpec((D, D), lambda i: (0, 0)),
        pl.BlockSpec((D, D), lambda i: (0, 0)),
        pl.BlockSpec((D_IN, D), lambda i: (0, 0)),
        pl.BlockSpec((1, D), lambda i: (0, 0)),
    ],
    out_specs=pl.BlockSpec((BLK, D), lambda i: (i, 0)),
    out_shape=jax.ShapeDtypeStruct((N, D), jnp.float32),
)


def _tc_post_body(agg_ref, h1_ref, batch_ref, wr_ref, wo2_ref, b2_ref,
                  wih_ref, whh_ref, bih_ref, bhh_ref, wd_ref, bd_ref,
                  wo_ref, bo_ref, out_ref, h_s, e_s):
    seg = lax.broadcasted_iota(jnp.int32, (1, G), 1)
    bg = bih_ref[...] + bhh_ref[...]

    q_star = jnp.zeros((G, 2 * D), jnp.float32)
    hh = jnp.zeros((G, D), jnp.float32)
    cc = jnp.zeros((G, D), jnp.float32)
    for si in range(STEPS):
        gates = _dot(q_star, wih_ref[...]) + _dot(hh, whh_ref[...]) + bg
        gi = jax.nn.sigmoid(gates[:, 0:D])
        gf = jax.nn.sigmoid(gates[:, D:2 * D])
        gg = jnp.tanh(gates[:, 2 * D:3 * D])
        go = jax.nn.sigmoid(gates[:, 3 * D:4 * D])
        cc = gf * cc + gi * gg
        hh = go * jnp.tanh(cc)

        def pass_a(i, emax_c):
            blk = pl.ds(i * BLK, BLK)
            if si == 0:
                aggs = agg_ref[0, blk, :] + agg_ref[1, blk, :]
                hb = jnp.maximum(
                    _dot(aggs, wr_ref[...])
                    + _dot(h1_ref[blk, :], wo2_ref[...]) + b2_ref[...], 0.0)
                h_s[blk, :] = hb
            else:
                hb = h_s[blk, :]
            mb = batch_ref[blk, :] == seg
            qb = _dot_hi(mb.astype(jnp.float32), hh)
            e = jnp.sum(hb * qb, axis=1, keepdims=True)
            e_s[blk, :] = e
            blkmax = jnp.max(jnp.where(mb, e, -jnp.inf), axis=0,
                             keepdims=True)
            return jnp.maximum(emax_c, blkmax)

        emax = lax.fori_loop(0, NBLK, pass_a,
                             jnp.full((1, G), -jnp.inf, jnp.float32))

        def pass_b(i, carry):
            den_c, run_c = carry
            blk = pl.ds(i * BLK, BLK)
            mb = batch_ref[blk, :] == seg
            m = mb.astype(jnp.float32)
            hb = h_s[blk, :]
            e = e_s[blk, :]
            emaxb = jnp.sum(jnp.where(mb, emax, 0.0), axis=1, keepdims=True)
            eexp = jnp.exp(e - emaxb)
            den_c = den_c + _dot_t_hi(m, eexp)
            run_c = run_c + _dot_t_hi(m, eexp * hb)
            return den_c, run_c

        den, run = lax.fori_loop(
            0, NBLK, pass_b,
            (jnp.zeros((G, 1), jnp.float32), jnp.zeros((G, D), jnp.float32)))
        r = run / jnp.where(den > 0.0, den, 1.0)
        q_star = jnp.concatenate([hh, r], axis=1)

    o = jnp.maximum(_dot(q_star, wd_ref[...]) + bd_ref[...], 0.0)
    out_ref[...] = _dot(o, wo_ref[...]) + bo_ref[...]


_tc_post = pl.pallas_call(
    _tc_post_body,
    out_shape=jax.ShapeDtypeStruct((G, 1), jnp.float32),
    scratch_shapes=[pltpu.VMEM((N, D), jnp.float32),
                    pltpu.VMEM((N, 1), jnp.float32)],
)


def kernel(x, edge_index, batch, W_rel1, W_root1, b1, W_rel2, W_root2, b2,
           W_ih, W_hh, b_ih, b_hh, W_dense, b_dense, W_out, b_out):
    src = edge_index[0].astype(jnp.int32)
    dst = edge_index[1].astype(jnp.int32)
    pad = E_PAD - E
    src_pad = jnp.concatenate([src, jnp.zeros((pad,), jnp.int32)]).reshape(
        NW * CHUNKS_PER_W, CHUNK)
    dst_pad = jnp.concatenate([dst, jnp.full((pad,), N, jnp.int32)]).reshape(
        NW * CHUNKS_PER_W, CHUNK)
    zeros_d = jnp.zeros((N_ACC, D), jnp.float32)
    xtab = jnp.stack([jnp.pad(x[:, :D], ((0, N_ACC - N), (0, 0))),
                      jnp.pad(x[:, D:], ((0, N_ACC - N), (0, 0)))])

    agg1 = _get_sc_scatter_spmem(2)(xtab, zeros_d, src_pad, dst_pad)
    h1 = _tc_l1(agg1, x, W_rel1[:D], W_rel1[D:], W_root1, b1.reshape(1, D))
    h1_pad = jnp.pad(h1, ((0, N_ACC - N), (0, 0)))
    agg2 = _get_sc_scatter_spmem(1)(h1_pad.reshape(1, N_ACC, D), zeros_d,
                                    src_pad, dst_pad).reshape(NC, N_ACC, D)
    out = _tc_post(agg2, h1, batch.astype(jnp.int32).reshape(N, 1),
                   W_rel2, W_root2, b2.reshape(1, D),
                   W_ih.T, W_hh.T, b_ih.reshape(1, 4 * D),
                   b_hh.reshape(1, 4 * D), W_dense, b_dense.reshape(1, D),
                   W_out, b_out.reshape(1, 1))
    return out.reshape(G)
